# initial kernel scaffold (unmeasured)
import functools

import jax
import jax.numpy as jnp
import numpy as np
from jax import lax
from jax.experimental import pallas as pl
from jax.experimental.pallas import tpu as pltpu

N_DEV = 32
M = 8192
N = 4096
CHUNK = M // N_DEV


def _logical_order():
    out = []
    for z in range(4):
        for yi, y in enumerate(range(4)):
            row = [(0, y, z), (1, y, z)]
            if yi % 2:
                row = row[::-1]
            out += row
    return out


def _ring_order():
    path_yz = [
        (0, 0), (0, 1), (0, 2), (0, 3),
        (1, 3), (1, 2), (1, 1), (1, 0),
        (2, 0), (2, 1), (2, 2), (2, 3),
        (3, 3), (3, 2), (3, 1), (3, 0),
    ]
    ring = [(0, y, z) for (y, z) in path_yz]
    ring += [(1, y, z) for (y, z) in reversed(path_yz)]
    return ring


_LOG = _logical_order()
_RING = _ring_order()
_COORD_TO_LOG = {c: i for i, c in enumerate(_LOG)}
_COORD_TO_RING = {c: i for i, c in enumerate(_RING)}
_RING_POS = [_COORD_TO_RING[c] for c in _LOG]
_RIGHT = [_COORD_TO_LOG[_RING[(_COORD_TO_RING[c] + 1) % N_DEV]] for c in _LOG]
_LEFT = [_COORD_TO_LOG[_RING[(_COORD_TO_RING[c] - 1) % N_DEV]] for c in _LOG]


def kernel(x, w_mat):
    m, k_per = x.shape
    _, n = w_mat.shape
    assert m == M and n == N

    ring_pos_arr = jnp.array(_RING_POS, dtype=jnp.int32)
    right_arr = jnp.array(_RIGHT, dtype=jnp.int32)
    left_arr = jnp.array(_LEFT, dtype=jnp.int32)

    def body(x_ref, w_ref, out_ref, comm_ref, ag_ref,
             rs_send_sems, rs_recv_sems, ag_send_sems, ag_recv_sems,
             copy_sem, rs_credit, ag_credit):
        d = lax.axis_index("i")
        r = jnp.take(ring_pos_arr, d)
        right = jnp.take(right_arr, d)
        left = jnp.take(left_arr, d)

        def partial_chunk(c):
            a = x_ref[pl.ds(c * CHUNK, CHUNK), :]
            return jnp.dot(a, w_ref[:, :], preferred_element_type=jnp.float32)

        barrier_sem = pltpu.get_barrier_semaphore()
        pl.semaphore_signal(barrier_sem, inc=1, device_id=(left,),
                            device_id_type=pl.DeviceIdType.MESH)
        pl.semaphore_signal(barrier_sem, inc=1, device_id=(right,),
                            device_id_type=pl.DeviceIdType.MESH)
        pl.semaphore_wait(barrier_sem, 2)

        comm_ref[0, :, :] = partial_chunk(r)
        for h in range(N_DEV - 1):
            send_slot = h % 2
            recv_slot = (h + 1) % 2
            if h >= 2:
                pl.semaphore_wait(rs_credit, 1)
            rdma = pltpu.make_async_remote_copy(
                src_ref=comm_ref.at[send_slot],
                dst_ref=comm_ref.at[recv_slot],
                send_sem=rs_send_sems.at[send_slot],
                recv_sem=rs_recv_sems.at[recv_slot],
                device_id=(right,),
                device_id_type=pl.DeviceIdType.MESH,
            )
            rdma.start()
            rdma.wait()
            c = (r - h - 1) % N_DEV
            comm_ref[recv_slot, :, :] = comm_ref[recv_slot, :, :] + partial_chunk(c)
            if h <= N_DEV - 4:
                pl.semaphore_signal(rs_credit, inc=1, device_id=(left,),
                                    device_id_type=pl.DeviceIdType.MESH)

        red = comm_ref[(N_DEV - 1) % 2, :, :]
        g = red * jax.nn.sigmoid(red)
        my_chunk = (r + 1) % N_DEV
        ag_ref[0, :, :] = g
        own_copy = pltpu.make_async_copy(
            ag_ref.at[0], out_ref.at[pl.ds(my_chunk * CHUNK, CHUNK), :], copy_sem,
        )
        own_copy.start()
        own_copy.wait()

        for t in range(N_DEV - 1):
            send_slot = t % 2
            recv_slot = (t + 1) % 2
            if t >= 2:
                pl.semaphore_wait(ag_credit, 1)
            rdma = pltpu.make_async_remote_copy(
                src_ref=ag_ref.at[send_slot],
                dst_ref=ag_ref.at[recv_slot],
                send_sem=ag_send_sems.at[send_slot],
                recv_sem=ag_recv_sems.at[recv_slot],
                device_id=(right,),
                device_id_type=pl.DeviceIdType.MESH,
            )
            rdma.start()
            rdma.wait()
            origin = (my_chunk - t - 1) % N_DEV
            cp = pltpu.make_async_copy(
                ag_ref.at[recv_slot],
                out_ref.at[pl.ds(origin * CHUNK, CHUNK), :],
                copy_sem,
            )
            cp.start()
            cp.wait()
            if t <= N_DEV - 4:
                pl.semaphore_signal(ag_credit, inc=1, device_id=(left,),
                                    device_id_type=pl.DeviceIdType.MESH)

    return pl.pallas_call(
        body,
        out_shape=jax.ShapeDtypeStruct((M, N), jnp.float32),
        in_specs=[
            pl.BlockSpec(memory_space=pltpu.VMEM),
            pl.BlockSpec(memory_space=pltpu.VMEM),
        ],
        out_specs=pl.BlockSpec(memory_space=pltpu.MemorySpace.ANY),
        scratch_shapes=[
            pltpu.VMEM((2, CHUNK, N), jnp.float32),
            pltpu.VMEM((2, CHUNK, N), jnp.float32),
            pltpu.SemaphoreType.DMA((2,)),
            pltpu.SemaphoreType.DMA((2,)),
            pltpu.SemaphoreType.DMA((2,)),
            pltpu.SemaphoreType.DMA((2,)),
            pltpu.SemaphoreType.DMA,
            pltpu.SemaphoreType.REGULAR,
            pltpu.SemaphoreType.REGULAR,
        ],
        compiler_params=pltpu.CompilerParams(collective_id=0),
    )(x, w_mat)


# baseline (device time: 3084971 ns/iter reference)
import functools

import jax
import jax.numpy as jnp
import numpy as np
from jax import lax
from jax.experimental import pallas as pl
from jax.experimental.pallas import tpu as pltpu

N_DEV = 32
M = 8192
N = 4096
CHUNK = M // N_DEV


def _logical_order():
    out = []
    for z in range(4):
        for yi, y in enumerate(range(4)):
            row = [(0, y, z), (1, y, z)]
            if yi % 2:
                row = row[::-1]
            out += row
    return out


def _ring_order():
    path_yz = [
        (0, 0), (0, 1), (0, 2), (0, 3),
        (1, 3), (1, 2), (1, 1), (1, 0),
        (2, 0), (2, 1), (2, 2), (2, 3),
        (3, 3), (3, 2), (3, 1), (3, 0),
    ]
    ring = [(0, y, z) for (y, z) in path_yz]
    ring += [(1, y, z) for (y, z) in reversed(path_yz)]
    return ring


_LOG = _logical_order()
_RING = _ring_order()
_COORD_TO_LOG = {c: i for i, c in enumerate(_LOG)}
_COORD_TO_RING = {c: i for i, c in enumerate(_RING)}
_RING_POS = [_COORD_TO_RING[c] for c in _LOG]
_RIGHT = [_COORD_TO_LOG[_RING[(_COORD_TO_RING[c] + 1) % N_DEV]] for c in _LOG]
_LEFT = [_COORD_TO_LOG[_RING[(_COORD_TO_RING[c] - 1) % N_DEV]] for c in _LOG]


def kernel(x, w_mat):
    m, k_per = x.shape
    _, n = w_mat.shape
    assert m == M and n == N

    ring_pos_arr = jnp.array(_RING_POS, dtype=jnp.int32)
    right_arr = jnp.array(_RIGHT, dtype=jnp.int32)
    left_arr = jnp.array(_LEFT, dtype=jnp.int32)

    def body(ring_pos_ref, right_ref, left_ref, x_ref, w_ref, out_ref,
             comm_ref, ag_ref,
             rs_send_sems, rs_recv_sems, ag_send_sems, ag_recv_sems,
             copy_sem, rs_credit, ag_credit):
        d = lax.axis_index("i")
        r = ring_pos_ref[d]
        right = right_ref[d]
        left = left_ref[d]

        def partial_chunk(c):
            a = x_ref[pl.ds(c * CHUNK, CHUNK), :]
            return jnp.dot(a, w_ref[:, :], preferred_element_type=jnp.float32)

        barrier_sem = pltpu.get_barrier_semaphore()
        pl.semaphore_signal(barrier_sem, inc=1, device_id=(left,),
                            device_id_type=pl.DeviceIdType.MESH)
        pl.semaphore_signal(barrier_sem, inc=1, device_id=(right,),
                            device_id_type=pl.DeviceIdType.MESH)
        pl.semaphore_wait(barrier_sem, 2)

        comm_ref[0, :, :] = partial_chunk(r)
        for h in range(N_DEV - 1):
            send_slot = h % 2
            recv_slot = (h + 1) % 2
            if h >= 2:
                pl.semaphore_wait(rs_credit, 1)
            rdma = pltpu.make_async_remote_copy(
                src_ref=comm_ref.at[send_slot],
                dst_ref=comm_ref.at[recv_slot],
                send_sem=rs_send_sems.at[send_slot],
                recv_sem=rs_recv_sems.at[recv_slot],
                device_id=(right,),
                device_id_type=pl.DeviceIdType.MESH,
            )
            rdma.start()
            rdma.wait()
            c = (r - h - 1) % N_DEV
            comm_ref[recv_slot, :, :] = comm_ref[recv_slot, :, :] + partial_chunk(c)
            if h <= N_DEV - 4:
                pl.semaphore_signal(rs_credit, inc=1, device_id=(left,),
                                    device_id_type=pl.DeviceIdType.MESH)

        red = comm_ref[(N_DEV - 1) % 2, :, :]
        g = red * jax.nn.sigmoid(red)
        my_chunk = (r + 1) % N_DEV
        ag_ref[0, :, :] = g
        own_copy = pltpu.make_async_copy(
            ag_ref.at[0], out_ref.at[pl.ds(my_chunk * CHUNK, CHUNK), :], copy_sem,
        )
        own_copy.start()
        own_copy.wait()

        for t in range(N_DEV - 1):
            send_slot = t % 2
            recv_slot = (t + 1) % 2
            if t >= 2:
                pl.semaphore_wait(ag_credit, 1)
            rdma = pltpu.make_async_remote_copy(
                src_ref=ag_ref.at[send_slot],
                dst_ref=ag_ref.at[recv_slot],
                send_sem=ag_send_sems.at[send_slot],
                recv_sem=ag_recv_sems.at[recv_slot],
                device_id=(right,),
                device_id_type=pl.DeviceIdType.MESH,
            )
            rdma.start()
            rdma.wait()
            origin = (my_chunk - t - 1) % N_DEV
            cp = pltpu.make_async_copy(
                ag_ref.at[recv_slot],
                out_ref.at[pl.ds(origin * CHUNK, CHUNK), :],
                copy_sem,
            )
            cp.start()
            cp.wait()
            if t <= N_DEV - 4:
                pl.semaphore_signal(ag_credit, inc=1, device_id=(left,),
                                    device_id_type=pl.DeviceIdType.MESH)

    return pl.pallas_call(
        body,
        out_shape=jax.ShapeDtypeStruct((M, N), jnp.float32),
        in_specs=[
            pl.BlockSpec(memory_space=pltpu.MemorySpace.SMEM),
            pl.BlockSpec(memory_space=pltpu.MemorySpace.SMEM),
            pl.BlockSpec(memory_space=pltpu.MemorySpace.SMEM),
            pl.BlockSpec(memory_space=pltpu.VMEM),
            pl.BlockSpec(memory_space=pltpu.VMEM),
        ],
        out_specs=pl.BlockSpec(memory_space=pltpu.MemorySpace.HBM),
        scratch_shapes=[
            pltpu.VMEM((2, CHUNK, N), jnp.float32),
            pltpu.VMEM((2, CHUNK, N), jnp.float32),
            pltpu.SemaphoreType.DMA((2,)),
            pltpu.SemaphoreType.DMA((2,)),
            pltpu.SemaphoreType.DMA((2,)),
            pltpu.SemaphoreType.DMA((2,)),
            pltpu.SemaphoreType.DMA,
            pltpu.SemaphoreType.REGULAR,
            pltpu.SemaphoreType.REGULAR,
        ],
        compiler_params=pltpu.CompilerParams(collective_id=0),
    )(ring_pos_arr, right_arr, left_arr, x, w_mat)


# device time: 1722086 ns/iter; 1.7914x vs baseline; 1.7914x over previous
import jax
import jax.numpy as jnp
import numpy as np
from jax import lax
from jax.experimental import pallas as pl
from jax.experimental.pallas import tpu as pltpu

N_DEV = 32
M = 8192
N = 4096
CHUNK = M // N_DEV
HALF = CHUNK // 2


def _logical_order():
    out = []
    for z in range(4):
        for yi, y in enumerate(range(4)):
            row = [(0, y, z), (1, y, z)]
            if yi % 2:
                row = row[::-1]
            out += row
    return out


def _ring_order():
    path_yz = [
        (0, 0), (0, 1), (0, 2), (0, 3),
        (1, 3), (1, 2), (1, 1), (1, 0),
        (2, 0), (2, 1), (2, 2), (2, 3),
        (3, 3), (3, 2), (3, 1), (3, 0),
    ]
    ring = [(0, y, z) for (y, z) in path_yz]
    ring += [(1, y, z) for (y, z) in reversed(path_yz)]
    return ring


_LOG = _logical_order()
_RING = _ring_order()
_COORD_TO_LOG = {c: i for i, c in enumerate(_LOG)}
_COORD_TO_RING = {c: i for i, c in enumerate(_RING)}
_RING_POS = [_COORD_TO_RING[c] for c in _LOG]
_RIGHT = [_COORD_TO_LOG[_RING[(_COORD_TO_RING[c] + 1) % N_DEV]] for c in _LOG]
_LEFT = [_COORD_TO_LOG[_RING[(_COORD_TO_RING[c] - 1) % N_DEV]] for c in _LOG]


def kernel(x, w_mat):
    m, k_per = x.shape
    _, n = w_mat.shape
    assert m == M and n == N

    ring_pos_arr = jnp.array(_RING_POS, dtype=jnp.int32)
    right_arr = jnp.array(_RIGHT, dtype=jnp.int32)
    left_arr = jnp.array(_LEFT, dtype=jnp.int32)

    def body(ring_pos_ref, right_ref, left_ref, x_ref, w_ref, out_ref,
             cw_ref, ccw_ref, agcw_ref, agccw_ref,
             cw_ssem, cw_rsem, ccw_ssem, ccw_rsem,
             agcw_ssem, agcw_rsem, agccw_ssem, agccw_rsem,
             copy_sem, cred_cw, cred_ccw, agcred_cw, agcred_ccw):
        d = lax.axis_index("i")
        r = ring_pos_ref[d]
        right = right_ref[d]
        left = left_ref[d]

        def partial_top(c):
            a = x_ref[pl.ds(c * CHUNK, HALF), :]
            return jnp.dot(a, w_ref[:, :], preferred_element_type=jnp.float32)

        def partial_bot(c):
            a = x_ref[pl.ds(c * CHUNK + HALF, HALF), :]
            return jnp.dot(a, w_ref[:, :], preferred_element_type=jnp.float32)

        barrier_sem = pltpu.get_barrier_semaphore()
        pl.semaphore_signal(barrier_sem, inc=1, device_id=(left,),
                            device_id_type=pl.DeviceIdType.MESH)
        pl.semaphore_signal(barrier_sem, inc=1, device_id=(right,),
                            device_id_type=pl.DeviceIdType.MESH)
        pl.semaphore_wait(barrier_sem, 2)

        cw_ref[0, :, :] = partial_top(r)
        ccw_ref[0, :, :] = partial_bot(r)
        for h in range(N_DEV - 1):
            s = h % 2
            rs = (h + 1) % 2
            if h >= 2:
                pl.semaphore_wait(cred_cw, 1)
                pl.semaphore_wait(cred_ccw, 1)
            rd_cw = pltpu.make_async_remote_copy(
                src_ref=cw_ref.at[s], dst_ref=cw_ref.at[rs],
                send_sem=cw_ssem.at[s], recv_sem=cw_rsem.at[rs],
                device_id=(right,), device_id_type=pl.DeviceIdType.MESH,
            )
            rd_ccw = pltpu.make_async_remote_copy(
                src_ref=ccw_ref.at[s], dst_ref=ccw_ref.at[rs],
                send_sem=ccw_ssem.at[s], recv_sem=ccw_rsem.at[rs],
                device_id=(left,), device_id_type=pl.DeviceIdType.MESH,
            )
            rd_cw.start()
            rd_ccw.start()
            rd_cw.wait()
            rd_ccw.wait()
            c_cw = (r - h - 1) % N_DEV
            c_ccw = (r + h + 1) % N_DEV
            cw_ref[rs, :, :] = cw_ref[rs, :, :] + partial_top(c_cw)
            ccw_ref[rs, :, :] = ccw_ref[rs, :, :] + partial_bot(c_ccw)
            if h <= N_DEV - 4:
                pl.semaphore_signal(cred_cw, inc=1, device_id=(left,),
                                    device_id_type=pl.DeviceIdType.MESH)
                pl.semaphore_signal(cred_ccw, inc=1, device_id=(right,),
                                    device_id_type=pl.DeviceIdType.MESH)

        f = (N_DEV - 1) % 2
        top = cw_ref[f, :, :]
        bot = ccw_ref[f, :, :]
        chunk_cw = (r + 1) % N_DEV
        chunk_ccw = (r - 1) % N_DEV
        agcw_ref[0, :, :] = top * jax.nn.sigmoid(top)
        agccw_ref[0, :, :] = bot * jax.nn.sigmoid(bot)
        cp1 = pltpu.make_async_copy(
            agcw_ref.at[0], out_ref.at[pl.ds(chunk_cw * CHUNK, HALF), :],
            copy_sem)
        cp1.start()
        cp1.wait()
        cp2 = pltpu.make_async_copy(
            agccw_ref.at[0],
            out_ref.at[pl.ds(chunk_ccw * CHUNK + HALF, HALF), :],
            copy_sem)
        cp2.start()
        cp2.wait()

        for t in range(N_DEV - 1):
            s = t % 2
            rs = (t + 1) % 2
            if t >= 2:
                pl.semaphore_wait(agcred_cw, 1)
                pl.semaphore_wait(agcred_ccw, 1)
            rd_cw = pltpu.make_async_remote_copy(
                src_ref=agcw_ref.at[s], dst_ref=agcw_ref.at[rs],
                send_sem=agcw_ssem.at[s], recv_sem=agcw_rsem.at[rs],
                device_id=(right,), device_id_type=pl.DeviceIdType.MESH,
            )
            rd_ccw = pltpu.make_async_remote_copy(
                src_ref=agccw_ref.at[s], dst_ref=agccw_ref.at[rs],
                send_sem=agccw_ssem.at[s], recv_sem=agccw_rsem.at[rs],
                device_id=(left,), device_id_type=pl.DeviceIdType.MESH,
            )
            rd_cw.start()
            rd_ccw.start()
            rd_cw.wait()
            rd_ccw.wait()
            o_cw = (r - t) % N_DEV
            o_ccw = (r + t) % N_DEV
            cp1 = pltpu.make_async_copy(
                agcw_ref.at[rs], out_ref.at[pl.ds(o_cw * CHUNK, HALF), :],
                copy_sem)
            cp1.start()
            cp1.wait()
            cp2 = pltpu.make_async_copy(
                agccw_ref.at[rs],
                out_ref.at[pl.ds(o_ccw * CHUNK + HALF, HALF), :],
                copy_sem)
            cp2.start()
            cp2.wait()
            if t <= N_DEV - 4:
                pl.semaphore_signal(agcred_cw, inc=1, device_id=(left,),
                                    device_id_type=pl.DeviceIdType.MESH)
                pl.semaphore_signal(agcred_ccw, inc=1, device_id=(right,),
                                    device_id_type=pl.DeviceIdType.MESH)

    return pl.pallas_call(
        body,
        out_shape=jax.ShapeDtypeStruct((M, N), jnp.float32),
        in_specs=[
            pl.BlockSpec(memory_space=pltpu.MemorySpace.SMEM),
            pl.BlockSpec(memory_space=pltpu.MemorySpace.SMEM),
            pl.BlockSpec(memory_space=pltpu.MemorySpace.SMEM),
            pl.BlockSpec(memory_space=pltpu.VMEM),
            pl.BlockSpec(memory_space=pltpu.VMEM),
        ],
        out_specs=pl.BlockSpec(memory_space=pltpu.MemorySpace.HBM),
        scratch_shapes=[
            pltpu.VMEM((2, HALF, N), jnp.float32),
            pltpu.VMEM((2, HALF, N), jnp.float32),
            pltpu.VMEM((2, HALF, N), jnp.float32),
            pltpu.VMEM((2, HALF, N), jnp.float32),
            pltpu.SemaphoreType.DMA((2,)),
            pltpu.SemaphoreType.DMA((2,)),
            pltpu.SemaphoreType.DMA((2,)),
            pltpu.SemaphoreType.DMA((2,)),
            pltpu.SemaphoreType.DMA((2,)),
            pltpu.SemaphoreType.DMA((2,)),
            pltpu.SemaphoreType.DMA((2,)),
            pltpu.SemaphoreType.DMA((2,)),
            pltpu.SemaphoreType.DMA,
            pltpu.SemaphoreType.REGULAR,
            pltpu.SemaphoreType.REGULAR,
            pltpu.SemaphoreType.REGULAR,
            pltpu.SemaphoreType.REGULAR,
        ],
        compiler_params=pltpu.CompilerParams(collective_id=0),
    )(ring_pos_arr, right_arr, left_arr, x, w_mat)


# device time: 1617410 ns/iter; 1.9074x vs baseline; 1.0647x over previous
import jax
import jax.numpy as jnp
import numpy as np
from jax import lax
from jax.experimental import pallas as pl
from jax.experimental.pallas import tpu as pltpu

N_DEV = 32
M = 8192
N = 4096
CHUNK = M // N_DEV
HALF = CHUNK // 2


def _logical_order():
    out = []
    for z in range(4):
        for yi, y in enumerate(range(4)):
            row = [(0, y, z), (1, y, z)]
            if yi % 2:
                row = row[::-1]
            out += row
    return out


def _ring_order():
    path_yz = [
        (0, 0), (0, 1), (0, 2), (0, 3),
        (1, 3), (1, 2), (1, 1), (1, 0),
        (2, 0), (2, 1), (2, 2), (2, 3),
        (3, 3), (3, 2), (3, 1), (3, 0),
    ]
    ring = [(0, y, z) for (y, z) in path_yz]
    ring += [(1, y, z) for (y, z) in reversed(path_yz)]
    return ring


_LOG = _logical_order()
_RING = _ring_order()
_COORD_TO_LOG = {c: i for i, c in enumerate(_LOG)}
_COORD_TO_RING = {c: i for i, c in enumerate(_RING)}
_RING_POS = [_COORD_TO_RING[c] for c in _LOG]
_RIGHT = [_COORD_TO_LOG[_RING[(_COORD_TO_RING[c] + 1) % N_DEV]] for c in _LOG]
_LEFT = [_COORD_TO_LOG[_RING[(_COORD_TO_RING[c] - 1) % N_DEV]] for c in _LOG]


def kernel(x, w_mat):
    m, k_per = x.shape
    _, n = w_mat.shape
    assert m == M and n == N

    ring_pos_arr = jnp.array(_RING_POS, dtype=jnp.int32)
    right_arr = jnp.array(_RIGHT, dtype=jnp.int32)
    left_arr = jnp.array(_LEFT, dtype=jnp.int32)

    def body(ring_pos_ref, right_ref, left_ref, x_ref, w_ref, out_ref,
             cw_ref, ccw_ref, agcw_ref, agccw_ref,
             cw_ssem, cw_rsem, ccw_ssem, ccw_rsem,
             agcw_ssem, agcw_rsem, agccw_ssem, agccw_rsem,
             copy_sem, copy_sem2, cred_cw, cred_ccw, agcred_cw, agcred_ccw):
        d = lax.axis_index("i")
        r = ring_pos_ref[d]
        right = right_ref[d]
        left = left_ref[d]

        def partial_top(c):
            a = x_ref[pl.ds(c * CHUNK, HALF), :]
            return jnp.dot(a, w_ref[:, :], preferred_element_type=jnp.float32)

        def partial_bot(c):
            a = x_ref[pl.ds(c * CHUNK + HALF, HALF), :]
            return jnp.dot(a, w_ref[:, :], preferred_element_type=jnp.float32)

        barrier_sem = pltpu.get_barrier_semaphore()
        pl.semaphore_signal(barrier_sem, inc=1, device_id=(left,),
                            device_id_type=pl.DeviceIdType.MESH)
        pl.semaphore_signal(barrier_sem, inc=1, device_id=(right,),
                            device_id_type=pl.DeviceIdType.MESH)
        pl.semaphore_wait(barrier_sem, 2)

        cw_ref[0, :, :] = partial_top(r)
        ccw_ref[0, :, :] = partial_bot(r)
        for h in range(N_DEV - 1):
            s = h % 2
            rs = (h + 1) % 2
            if h >= 2:
                pl.semaphore_wait(cred_cw, 1)
                pl.semaphore_wait(cred_ccw, 1)
            rd_cw = pltpu.make_async_remote_copy(
                src_ref=cw_ref.at[s], dst_ref=cw_ref.at[rs],
                send_sem=cw_ssem.at[s], recv_sem=cw_rsem.at[rs],
                device_id=(right,), device_id_type=pl.DeviceIdType.MESH,
            )
            rd_ccw = pltpu.make_async_remote_copy(
                src_ref=ccw_ref.at[s], dst_ref=ccw_ref.at[rs],
                send_sem=ccw_ssem.at[s], recv_sem=ccw_rsem.at[rs],
                device_id=(left,), device_id_type=pl.DeviceIdType.MESH,
            )
            rd_cw.start()
            rd_ccw.start()
            p_cw = partial_top((r - h - 1) % N_DEV)
            p_ccw = partial_bot((r + h + 1) % N_DEV)
            rd_cw.wait()
            rd_ccw.wait()
            cw_ref[rs, :, :] = cw_ref[rs, :, :] + p_cw
            ccw_ref[rs, :, :] = ccw_ref[rs, :, :] + p_ccw
            if h <= N_DEV - 4:
                pl.semaphore_signal(cred_cw, inc=1, device_id=(left,),
                                    device_id_type=pl.DeviceIdType.MESH)
                pl.semaphore_signal(cred_ccw, inc=1, device_id=(right,),
                                    device_id_type=pl.DeviceIdType.MESH)

        f = (N_DEV - 1) % 2
        top = cw_ref[f, :, :]
        bot = ccw_ref[f, :, :]
        agcw_ref[0, :, :] = top * jax.nn.sigmoid(top)
        agccw_ref[0, :, :] = bot * jax.nn.sigmoid(bot)

        for t in range(N_DEV - 1):
            s = t % 3
            rs = (t + 1) % 3
            if t >= 3:
                pl.semaphore_wait(agcred_cw, 1)
                pl.semaphore_wait(agcred_ccw, 1)
            rd_cw = pltpu.make_async_remote_copy(
                src_ref=agcw_ref.at[s], dst_ref=agcw_ref.at[rs],
                send_sem=agcw_ssem.at[s], recv_sem=agcw_rsem.at[rs],
                device_id=(right,), device_id_type=pl.DeviceIdType.MESH,
            )
            rd_ccw = pltpu.make_async_remote_copy(
                src_ref=agccw_ref.at[s], dst_ref=agccw_ref.at[rs],
                send_sem=agccw_ssem.at[s], recv_sem=agccw_rsem.at[rs],
                device_id=(left,), device_id_type=pl.DeviceIdType.MESH,
            )
            rd_cw.start()
            rd_ccw.start()
            o_cw = (r + 1 - t) % N_DEV
            o_ccw = (r - 1 + t) % N_DEV
            cp1 = pltpu.make_async_copy(
                agcw_ref.at[s], out_ref.at[pl.ds(o_cw * CHUNK, HALF), :],
                copy_sem)
            cp2 = pltpu.make_async_copy(
                agccw_ref.at[s],
                out_ref.at[pl.ds(o_ccw * CHUNK + HALF, HALF), :],
                copy_sem2)
            cp1.start()
            cp2.start()
            cp1.wait()
            cp2.wait()
            rd_cw.wait()
            rd_ccw.wait()
            if 1 <= t <= N_DEV - 4:
                pl.semaphore_signal(agcred_cw, inc=1, device_id=(left,),
                                    device_id_type=pl.DeviceIdType.MESH)
                pl.semaphore_signal(agcred_ccw, inc=1, device_id=(right,),
                                    device_id_type=pl.DeviceIdType.MESH)

        fs = (N_DEV - 1) % 3
        o_cw = (r + 1 - (N_DEV - 1)) % N_DEV
        o_ccw = (r - 1 + (N_DEV - 1)) % N_DEV
        cp1 = pltpu.make_async_copy(
            agcw_ref.at[fs], out_ref.at[pl.ds(o_cw * CHUNK, HALF), :],
            copy_sem)
        cp2 = pltpu.make_async_copy(
            agccw_ref.at[fs],
            out_ref.at[pl.ds(o_ccw * CHUNK + HALF, HALF), :],
            copy_sem2)
        cp1.start()
        cp2.start()
        cp1.wait()
        cp2.wait()

    return pl.pallas_call(
        body,
        out_shape=jax.ShapeDtypeStruct((M, N), jnp.float32),
        in_specs=[
            pl.BlockSpec(memory_space=pltpu.MemorySpace.SMEM),
            pl.BlockSpec(memory_space=pltpu.MemorySpace.SMEM),
            pl.BlockSpec(memory_space=pltpu.MemorySpace.SMEM),
            pl.BlockSpec(memory_space=pltpu.VMEM),
            pl.BlockSpec(memory_space=pltpu.VMEM),
        ],
        out_specs=pl.BlockSpec(memory_space=pltpu.MemorySpace.HBM),
        scratch_shapes=[
            pltpu.VMEM((2, HALF, N), jnp.float32),
            pltpu.VMEM((2, HALF, N), jnp.float32),
            pltpu.VMEM((3, HALF, N), jnp.float32),
            pltpu.VMEM((3, HALF, N), jnp.float32),
            pltpu.SemaphoreType.DMA((2,)),
            pltpu.SemaphoreType.DMA((2,)),
            pltpu.SemaphoreType.DMA((2,)),
            pltpu.SemaphoreType.DMA((2,)),
            pltpu.SemaphoreType.DMA((3,)),
            pltpu.SemaphoreType.DMA((3,)),
            pltpu.SemaphoreType.DMA((3,)),
            pltpu.SemaphoreType.DMA((3,)),
            pltpu.SemaphoreType.DMA,
            pltpu.SemaphoreType.DMA,
            pltpu.SemaphoreType.REGULAR,
            pltpu.SemaphoreType.REGULAR,
            pltpu.SemaphoreType.REGULAR,
            pltpu.SemaphoreType.REGULAR,
        ],
        compiler_params=pltpu.CompilerParams(collective_id=0),
    )(ring_pos_arr, right_arr, left_arr, x, w_mat)


# device time: 1501542 ns/iter; 2.0545x vs baseline; 1.0772x over previous
import jax
import jax.numpy as jnp
import numpy as np
from jax import lax
from jax.experimental import pallas as pl
from jax.experimental.pallas import tpu as pltpu

N_DEV = 32
M = 8192
N = 4096
CHUNK = M // N_DEV
HALF = CHUNK // 2
SUB = HALF // 2
NSLOT = 3
NHOP = N_DEV - 1


def _logical_order():
    out = []
    for z in range(4):
        for yi, y in enumerate(range(4)):
            row = [(0, y, z), (1, y, z)]
            if yi % 2:
                row = row[::-1]
            out += row
    return out


def _ring_order():
    path_yz = [
        (0, 0), (0, 1), (0, 2), (0, 3),
        (1, 3), (1, 2), (1, 1), (1, 0),
        (2, 0), (2, 1), (2, 2), (2, 3),
        (3, 3), (3, 2), (3, 1), (3, 0),
    ]
    ring = [(0, y, z) for (y, z) in path_yz]
    ring += [(1, y, z) for (y, z) in reversed(path_yz)]
    return ring


_LOG = _logical_order()
_RING = _ring_order()
_COORD_TO_LOG = {c: i for i, c in enumerate(_LOG)}
_COORD_TO_RING = {c: i for i, c in enumerate(_RING)}
_RING_POS = [_COORD_TO_RING[c] for c in _LOG]
_RIGHT = [_COORD_TO_LOG[_RING[(_COORD_TO_RING[c] + 1) % N_DEV]] for c in _LOG]
_LEFT = [_COORD_TO_LOG[_RING[(_COORD_TO_RING[c] - 1) % N_DEV]] for c in _LOG]


def kernel(x, w_mat):
    m, k_per = x.shape
    _, n = w_mat.shape
    assert m == M and n == N

    ring_pos_arr = jnp.array(_RING_POS, dtype=jnp.int32)
    right_arr = jnp.array(_RIGHT, dtype=jnp.int32)
    left_arr = jnp.array(_LEFT, dtype=jnp.int32)

    def body(ring_pos_ref, right_ref, left_ref, x_ref, w_ref, out_ref,
             cw_ref, ccw_ref, agcw_ref, agccw_ref,
             cw_ssem, cw_rsem, ccw_ssem, ccw_rsem,
             agcw_ssem, agcw_rsem, agccw_ssem, agccw_rsem,
             cpcw_sem, cpccw_sem,
             cred_cw, cred_ccw, agcred_cw, agcred_ccw):
        d = lax.axis_index("i")
        r = ring_pos_ref[d]
        right = right_ref[d]
        left = left_ref[d]

        def partial_top(c):
            a = x_ref[pl.ds(c * CHUNK, HALF), :]
            return jnp.dot(a, w_ref[:, :], preferred_element_type=jnp.float32)

        def partial_bot(c):
            a = x_ref[pl.ds(c * CHUNK + HALF, HALF), :]
            return jnp.dot(a, w_ref[:, :], preferred_element_type=jnp.float32)

        def mk(buf, ssem, rsem, s_src, s_dst, q, tgt):
            return pltpu.make_async_remote_copy(
                src_ref=buf.at[s_src, q],
                dst_ref=buf.at[s_dst, q],
                send_sem=ssem.at[s_src, q],
                recv_sem=rsem.at[s_dst, q],
                device_id=(tgt,),
                device_id_type=pl.DeviceIdType.MESH,
            )

        barrier_sem = pltpu.get_barrier_semaphore()
        pl.semaphore_signal(barrier_sem, inc=1, device_id=(left,),
                            device_id_type=pl.DeviceIdType.MESH)
        pl.semaphore_signal(barrier_sem, inc=1, device_id=(right,),
                            device_id_type=pl.DeviceIdType.MESH)
        pl.semaphore_wait(barrier_sem, 2)

        p0_cw = partial_top(r)
        p0_ccw = partial_bot(r)
        for q in (0, 1):
            cw_ref[0, q, :, :] = p0_cw[q * SUB:(q + 1) * SUB, :]
            ccw_ref[0, q, :, :] = p0_ccw[q * SUB:(q + 1) * SUB, :]
        rs_cw = {}
        rs_ccw = {}
        for q in (0, 1):
            rd = mk(cw_ref, cw_ssem, cw_rsem, 0, 1, q, right)
            rd.start()
            rs_cw[(0, q)] = rd
            rd = mk(ccw_ref, ccw_ssem, ccw_rsem, 0, 1, q, left)
            rd.start()
            rs_ccw[(0, q)] = rd

        for h in range(NHOP):
            dst = (h + 1) % NSLOT
            nxt = (h + 2) % NSLOT
            last = h == NHOP - 1
            p_cw = partial_top((r - h - 1) % N_DEV)
            p_ccw = partial_bot((r + h + 1) % N_DEV)
            if (not last) and h + 1 >= NSLOT:
                pl.semaphore_wait(cred_cw, 1)
                pl.semaphore_wait(cred_ccw, 1)
            for q in (0, 1):
                rs_cw[(h, q)].wait_recv()
                cw_ref[dst, q, :, :] = (
                    cw_ref[dst, q, :, :] + p_cw[q * SUB:(q + 1) * SUB, :])
                if not last:
                    if h + 1 >= NSLOT:
                        rs_cw[(h - 2, q)].wait_send()
                    rd = mk(cw_ref, cw_ssem, cw_rsem, dst, nxt, q, right)
                    rd.start()
                    rs_cw[(h + 1, q)] = rd
                rs_ccw[(h, q)].wait_recv()
                ccw_ref[dst, q, :, :] = (
                    ccw_ref[dst, q, :, :] + p_ccw[q * SUB:(q + 1) * SUB, :])
                if not last:
                    if h + 1 >= NSLOT:
                        rs_ccw[(h - 2, q)].wait_send()
                    rd = mk(ccw_ref, ccw_ssem, ccw_rsem, dst, nxt, q, left)
                    rd.start()
                    rs_ccw[(h + 1, q)] = rd
            if h <= NHOP - 4:
                pl.semaphore_signal(cred_cw, inc=1, device_id=(left,),
                                    device_id_type=pl.DeviceIdType.MESH)
                pl.semaphore_signal(cred_ccw, inc=1, device_id=(right,),
                                    device_id_type=pl.DeviceIdType.MESH)
        for h in (NHOP - 3, NHOP - 2, NHOP - 1):
            for q in (0, 1):
                rs_cw[(h, q)].wait_send()
                rs_ccw[(h, q)].wait_send()

        fslot = NHOP % NSLOT
        chunk_cw = (r + 1) % N_DEV
        chunk_ccw = (r - 1) % N_DEV
        for q in (0, 1):
            t_half = cw_ref[fslot, q, :, :]
            b_half = ccw_ref[fslot, q, :, :]
            agcw_ref[0, q, :, :] = t_half * jax.nn.sigmoid(t_half)
            agccw_ref[0, q, :, :] = b_half * jax.nn.sigmoid(b_half)
        ag_cw = {}
        ag_ccw = {}
        for q in (0, 1):
            rd = mk(agcw_ref, agcw_ssem, agcw_rsem, 0, 1, q, right)
            rd.start()
            ag_cw[(0, q)] = rd
            rd = mk(agccw_ref, agccw_ssem, agccw_rsem, 0, 1, q, left)
            rd.start()
            ag_ccw[(0, q)] = rd

        def store(buf, slot, q, row0, sem):
            cp = pltpu.make_async_copy(
                buf.at[slot, q],
                out_ref.at[pl.ds(row0 + q * SUB, SUB), :],
                sem.at[q],
            )
            cp.start()
            return cp

        cps = [store(agcw_ref, 0, q, chunk_cw * CHUNK, cpcw_sem)
               for q in (0, 1)]
        cps += [store(agccw_ref, 0, q, chunk_ccw * CHUNK + HALF, cpccw_sem)
                for q in (0, 1)]
        for cp in cps:
            cp.wait()

        for t in range(NHOP):
            dst = (t + 1) % NSLOT
            nxt = (t + 2) % NSLOT
            last = t == NHOP - 1
            if (not last) and t + 1 >= NSLOT:
                pl.semaphore_wait(agcred_cw, 1)
                pl.semaphore_wait(agcred_ccw, 1)
            for q in (0, 1):
                ag_cw[(t, q)].wait_recv()
                if not last:
                    if t + 1 >= NSLOT:
                        ag_cw[(t - 2, q)].wait_send()
                    rd = mk(agcw_ref, agcw_ssem, agcw_rsem, dst, nxt, q, right)
                    rd.start()
                    ag_cw[(t + 1, q)] = rd
                ag_ccw[(t, q)].wait_recv()
                if not last:
                    if t + 1 >= NSLOT:
                        ag_ccw[(t - 2, q)].wait_send()
                    rd = mk(agccw_ref, agccw_ssem, agccw_rsem, dst, nxt, q, left)
                    rd.start()
                    ag_ccw[(t + 1, q)] = rd
            o_cw = (r - t) % N_DEV
            o_ccw = (r + t) % N_DEV
            cps = [store(agcw_ref, dst, q, o_cw * CHUNK, cpcw_sem)
                   for q in (0, 1)]
            cps += [store(agccw_ref, dst, q, o_ccw * CHUNK + HALF, cpccw_sem)
                    for q in (0, 1)]
            for cp in cps:
                cp.wait()
            if t <= NHOP - 4:
                pl.semaphore_signal(agcred_cw, inc=1, device_id=(left,),
                                    device_id_type=pl.DeviceIdType.MESH)
                pl.semaphore_signal(agcred_ccw, inc=1, device_id=(right,),
                                    device_id_type=pl.DeviceIdType.MESH)
        for t in (NHOP - 3, NHOP - 2, NHOP - 1):
            for q in (0, 1):
                ag_cw[(t, q)].wait_send()
                ag_ccw[(t, q)].wait_send()

    return pl.pallas_call(
        body,
        out_shape=jax.ShapeDtypeStruct((M, N), jnp.float32),
        in_specs=[
            pl.BlockSpec(memory_space=pltpu.MemorySpace.SMEM),
            pl.BlockSpec(memory_space=pltpu.MemorySpace.SMEM),
            pl.BlockSpec(memory_space=pltpu.MemorySpace.SMEM),
            pl.BlockSpec(memory_space=pltpu.VMEM),
            pl.BlockSpec(memory_space=pltpu.VMEM),
        ],
        out_specs=pl.BlockSpec(memory_space=pltpu.MemorySpace.HBM),
        scratch_shapes=[
            pltpu.VMEM((NSLOT, 2, SUB, N), jnp.float32),
            pltpu.VMEM((NSLOT, 2, SUB, N), jnp.float32),
            pltpu.VMEM((NSLOT, 2, SUB, N), jnp.float32),
            pltpu.VMEM((NSLOT, 2, SUB, N), jnp.float32),
            pltpu.SemaphoreType.DMA((NSLOT, 2)),
            pltpu.SemaphoreType.DMA((NSLOT, 2)),
            pltpu.SemaphoreType.DMA((NSLOT, 2)),
            pltpu.SemaphoreType.DMA((NSLOT, 2)),
            pltpu.SemaphoreType.DMA((NSLOT, 2)),
            pltpu.SemaphoreType.DMA((NSLOT, 2)),
            pltpu.SemaphoreType.DMA((NSLOT, 2)),
            pltpu.SemaphoreType.DMA((NSLOT, 2)),
            pltpu.SemaphoreType.DMA((2,)),
            pltpu.SemaphoreType.DMA((2,)),
            pltpu.SemaphoreType.REGULAR,
            pltpu.SemaphoreType.REGULAR,
            pltpu.SemaphoreType.REGULAR,
            pltpu.SemaphoreType.REGULAR,
        ],
        compiler_params=pltpu.CompilerParams(collective_id=0),
    )(ring_pos_arr, right_arr, left_arr, x, w_mat)
